# baseline (speedup 1.0000x reference)
import functools

import jax
import jax.numpy as jnp
from jax import lax
from jax.experimental import pallas as pl
from jax.experimental.pallas import tpu as pltpu
from jax.experimental.pallas import tpu_sc as plsc

BATCH = 4096
HIST = 50
EMB_DIM = 128

NUM_CORES = 2
NUM_SUBCORES = 16
NW = NUM_CORES * NUM_SUBCORES
PER_W = BATCH // NW
SETS = 4
GROUP = 4
BPG = 1
NGROUP = PER_W // GROUP

_mesh = plsc.VectorSubcoreMesh(core_axis_name="c", subcore_axis_name="s")


@functools.partial(
    pl.kernel,
    out_type=jax.ShapeDtypeStruct((BATCH, HIST, EMB_DIM), jnp.float32),
    mesh=_mesh,
    scratch_types=[
        pltpu.VMEM((PER_W // BPG, BPG * HIST), jnp.int32),
        [pltpu.VMEM((GROUP, HIST, EMB_DIM), jnp.float32) for _ in range(SETS)],
        [pltpu.SemaphoreType.DMA for _ in range(2 * SETS)],
    ],
)
def _emb_gather(table_hbm, idx_hbm, out_hbm, idx_v, bufs, sems):
    wid = lax.axis_index("s") * NUM_CORES + lax.axis_index("c")
    base = wid * PER_W
    gsems = sems[:SETS]
    psems = sems[SETS:]

    pltpu.sync_copy(idx_hbm.at[pl.ds(base // BPG, PER_W // BPG)], idx_v)

    def gather_copies(g, s):
        return [
            pltpu.make_async_copy(
                table_hbm.at[idx_v.at[g * (GROUP // BPG) + j]],
                bufs[s].at[j],
                gsems[s])
            for j in range(GROUP // BPG)
        ]

    def put_copy(g, s):
        return pltpu.make_async_copy(
            bufs[s], out_hbm.at[pl.ds(base + g * GROUP, GROUP)], psems[s])

    def start_gathers(g, s):
        for c in gather_copies(g, s):
            c.start()

    def wait_gathers(g, s):
        for c in gather_copies(g, s):
            c.wait()

    for s in range(SETS):
        start_gathers(s, s)

    def body(u, carry):
        g0 = SETS * u
        for s in range(SETS):
            wait_gathers(g0 + s, s)
            put_copy(g0 + s, s).start()
        for s in range(SETS):
            put_copy(g0 + s, s).wait()
            start_gathers(g0 + s + SETS, s)
        return carry

    lax.fori_loop(0, NGROUP // SETS - 1, body, 0, unroll=False)

    gl = NGROUP - SETS
    for s in range(SETS):
        wait_gathers(gl + s, s)
        put_copy(gl + s, s).start()
    for s in range(SETS):
        put_copy(gl + s, s).wait()


def kernel(x, table):
    idx = x.reshape(BATCH // BPG, BPG * HIST).astype(jnp.int32)
    return _emb_gather(table, idx)


# speedup vs baseline: 1.7607x; 1.7607x over previous
import functools

import jax
import jax.numpy as jnp
from jax import lax
from jax.experimental import pallas as pl
from jax.experimental.pallas import tpu as pltpu
from jax.experimental.pallas import tpu_sc as plsc

BATCH = 4096
HIST = 50
EMB_DIM = 128

NUM_CORES = 2
NUM_SUBCORES = 16
NW = NUM_CORES * NUM_SUBCORES
PER_W = BATCH // NW
SETS = 5
NGROUP = HIST

_mesh = plsc.VectorSubcoreMesh(core_axis_name="c", subcore_axis_name="s")


@functools.partial(
    pl.kernel,
    out_type=jax.ShapeDtypeStruct((HIST, BATCH, EMB_DIM), jnp.float32),
    mesh=_mesh,
    scratch_types=[
        pltpu.VMEM((HIST, PER_W), jnp.int32),
        [pltpu.VMEM((PER_W, EMB_DIM), jnp.float32) for _ in range(SETS)],
        [pltpu.SemaphoreType.DMA for _ in range(2 * SETS)],
    ],
)
def _emb_gather(table_hbm, idx_hbm, out_hbm, idx_v, bufs, sems):
    wid = lax.axis_index("s") * NUM_CORES + lax.axis_index("c")
    base = pl.multiple_of(wid * PER_W, PER_W)
    gsems = sems[:SETS]
    psems = sems[SETS:]

    pltpu.sync_copy(idx_hbm.at[:, pl.ds(base, PER_W)], idx_v)

    def gather_copy(g, s):
        return pltpu.make_async_copy(
            table_hbm.at[idx_v.at[g]], bufs[s], gsems[s])

    def put_copy(g, s):
        return pltpu.make_async_copy(
            bufs[s], out_hbm.at[g, pl.ds(base, PER_W)], psems[s])

    for s in range(SETS):
        gather_copy(s, s).start()

    def body(u, carry):
        g0 = SETS * u
        for s in range(SETS):
            gather_copy(g0 + s, s).wait()
            put_copy(g0 + s, s).start()
        for s in range(SETS):
            put_copy(g0 + s, s).wait()
            gather_copy(g0 + s + SETS, s).start()
        return carry

    lax.fori_loop(0, NGROUP // SETS - 1, body, 0, unroll=False)

    gl = NGROUP - SETS
    for s in range(SETS):
        gather_copy(gl + s, s).wait()
        put_copy(gl + s, s).start()
    for s in range(SETS):
        put_copy(gl + s, s).wait()


def kernel(x, table):
    xt = x.T.astype(jnp.int32)
    out = _emb_gather(table, xt)
    return jnp.transpose(out, (1, 0, 2))
